# baseline (speedup 1.0000x reference)
import functools

import jax
import jax.numpy as jnp
from jax import lax
from jax.experimental import pallas as pl
from jax.experimental.pallas import tpu as pltpu
from jax.experimental.pallas import tpu_sc as plsc

_CHUNK = 128


@functools.lru_cache(maxsize=None)
def _make_kernel(B, D, n_chunks, b_per_w):
    info = plsc.get_sparse_core_info()
    NC = info.num_cores
    mesh = plsc.VectorSubcoreMesh(core_axis_name="c", subcore_axis_name="s")

    @functools.partial(
        pl.kernel,
        mesh=mesh,
        compiler_params=pltpu.CompilerParams(use_tc_tiling_on_sc=False),
        out_type=(
            jax.ShapeDtypeStruct((B, D), jnp.float32),
            jax.ShapeDtypeStruct((B, D), jnp.float32),
            jax.ShapeDtypeStruct((B, D), jnp.float32),
        ),
        scratch_types=[
            pltpu.VMEM((n_chunks, _CHUNK), jnp.int32),
            pltpu.VMEM((n_chunks, _CHUNK), jnp.int32),
            pltpu.VMEM((n_chunks, _CHUNK), jnp.int32),
            pltpu.VMEM((b_per_w, D), jnp.float32),
            pltpu.VMEM((b_per_w, D), jnp.float32),
            pltpu.VMEM((b_per_w, D), jnp.float32),
            pltpu.SemaphoreType.DMA,
            pltpu.SemaphoreType.DMA,
            pltpu.SemaphoreType.DMA,
        ],
    )
    def k(subj_hbm, rel_hbm, obj_hbm, etab_hbm, rtab_hbm,
          o_subj, o_rel, o_obj,
          si_v, ri_v, oi_v, sr_v, rr_v, or_v, sem_s, sem_r, sem_o):
        wid = lax.axis_index("s") * NC + lax.axis_index("c")
        base = wid * b_per_w
        pltpu.sync_copy(subj_hbm.at[wid], si_v)
        pltpu.sync_copy(rel_hbm.at[wid], ri_v)
        pltpu.sync_copy(obj_hbm.at[wid], oi_v)
        copies = []
        for j in range(n_chunks):
            dst = pl.ds(j * _CHUNK, _CHUNK)
            copies.append(pltpu.async_copy(etab_hbm.at[si_v.at[j]], sr_v.at[dst], sem_s))
            copies.append(pltpu.async_copy(etab_hbm.at[oi_v.at[j]], or_v.at[dst], sem_o))
            copies.append(pltpu.async_copy(rtab_hbm.at[ri_v.at[j]], rr_v.at[dst], sem_r))
        for c in copies:
            c.wait()
        pltpu.sync_copy(sr_v, o_subj.at[pl.ds(base, b_per_w)])
        pltpu.sync_copy(rr_v, o_rel.at[pl.ds(base, b_per_w)])
        pltpu.sync_copy(or_v, o_obj.at[pl.ds(base, b_per_w)])

    return k


def kernel(subj, rel, obj, entity_table, relation_table):
    B = subj.shape[0]
    D = entity_table.shape[1]
    info = plsc.get_sparse_core_info()
    NW = info.num_cores * info.num_subcores
    b_per_w = B // NW
    n_chunks = b_per_w // _CHUNK
    k = _make_kernel(B, D, n_chunks, b_per_w)
    subj_r = subj.reshape(NW, n_chunks, _CHUNK)
    rel_r = rel.reshape(NW, n_chunks, _CHUNK)
    obj_r = obj.reshape(NW, n_chunks, _CHUNK)
    return k(subj_r, rel_r, obj_r, entity_table, relation_table)
